# Initial kernel scaffold; baseline (speedup 1.0000x reference)
import jax
import jax.numpy as jnp
from jax.experimental import pallas as pl


def kernel(x, rel_table, type_table):
    raise NotImplementedError("write your pallas kernel here")


# speedup vs baseline: 4.9155x; 4.9155x over previous
import functools

import jax
import jax.numpy as jnp
from jax import lax
from jax.experimental import pallas as pl
from jax.experimental.pallas import tpu as pltpu
from jax.experimental.pallas import tpu_sc as plsc

B, L, F = 4096, 50, 10
D = 32
NT = 8
G = 128

NC, NS = 2, 16
NW = NC * NS

TYPE_G = B * L * NT // G
REL_G = B * L // G

K = 8
TYPE_SG = TYPE_G // K
REL_SG = REL_G // K

_MESH = plsc.VectorSubcoreMesh(core_axis_name="c", subcore_axis_name="s")


@functools.partial(
    pl.kernel,
    out_type=(
        jax.ShapeDtypeStruct((REL_G, G, D), jnp.float32),
        jax.ShapeDtypeStruct((TYPE_G, G, D), jnp.float32),
    ),
    mesh=_MESH,
    compiler_params=pltpu.CompilerParams(use_tc_tiling_on_sc=False),
    scratch_types=(
        pltpu.VMEM((K, G), jnp.int32),
        pltpu.VMEM((K, G, D), jnp.float32),
        pltpu.SemaphoreType.DMA,
    ),
)
def _sc_gather(rel_idx_hbm, type_idx_hbm, rel_tab_hbm, type_tab_hbm,
               rel_out_hbm, type_out_hbm, idx_v, rows_v, sem):
    wid = lax.axis_index("s") * NC + lax.axis_index("c")

    def run(idx_hbm, tab_hbm, out_hbm, n_sg):
        trip = (n_sg - wid + NW - 1) // NW

        def step(i, carry):
            g = (wid + i * NW) * K
            pltpu.sync_copy(idx_hbm.at[pl.ds(g, K)], idx_v)
            cps = [
                pltpu.async_copy(tab_hbm.at[idx_v.at[j]], rows_v.at[j], sem)
                for j in range(K)
            ]
            for cp in cps:
                cp.wait()
            pltpu.sync_copy(rows_v, out_hbm.at[pl.ds(g, K)])
            return carry

        lax.fori_loop(0, trip, step, 0)

    run(type_idx_hbm, type_tab_hbm, type_out_hbm, TYPE_SG)
    run(rel_idx_hbm, rel_tab_hbm, rel_out_hbm, REL_SG)


def kernel(x, rel_table, type_table):
    xf = x.reshape(B * L, F)
    rel_idx = xf[:, F - 1].reshape(REL_G, G)
    type_idx = xf[:, :NT].reshape(TYPE_G, G)
    rel_out, type_out = _sc_gather(rel_idx, type_idx, rel_table, type_table)
    return (rel_out.reshape(B, L, D), type_out.reshape(B, L, NT, D))


# speedup vs baseline: 5.0505x; 1.0275x over previous
import functools

import jax
import jax.numpy as jnp
from jax import lax
from jax.experimental import pallas as pl
from jax.experimental.pallas import tpu as pltpu
from jax.experimental.pallas import tpu_sc as plsc

B, L, F = 4096, 50, 10
D = 32
NT = 8
G = 128

NC, NS = 2, 16
NW = NC * NS

TYPE_G = B * L * NT // G
REL_G = B * L // G

K = 16
TYPE_SG = TYPE_G // K
REL_SG = REL_G // K

_MESH = plsc.VectorSubcoreMesh(core_axis_name="c", subcore_axis_name="s")


@functools.partial(
    pl.kernel,
    out_type=(
        jax.ShapeDtypeStruct((REL_G, G, D), jnp.float32),
        jax.ShapeDtypeStruct((TYPE_G, G, D), jnp.float32),
    ),
    mesh=_MESH,
    compiler_params=pltpu.CompilerParams(use_tc_tiling_on_sc=False),
    scratch_types=(
        pltpu.VMEM((K, G), jnp.int32),
        pltpu.VMEM((K, G, D), jnp.float32),
        pltpu.SemaphoreType.DMA,
    ),
)
def _sc_gather(rel_idx_hbm, type_idx_hbm, rel_tab_hbm, type_tab_hbm,
               rel_out_hbm, type_out_hbm, idx_v, rows_v, sem):
    wid = lax.axis_index("s") * NC + lax.axis_index("c")

    def run(idx_hbm, tab_hbm, out_hbm, n_sg):
        trip = (n_sg - wid + NW - 1) // NW

        def step(i, carry):
            g = (wid + i * NW) * K
            pltpu.sync_copy(idx_hbm.at[pl.ds(g, K)], idx_v)
            cps = [
                pltpu.async_copy(tab_hbm.at[idx_v.at[j]], rows_v.at[j], sem)
                for j in range(K)
            ]
            for cp in cps:
                cp.wait()
            pltpu.sync_copy(rows_v, out_hbm.at[pl.ds(g, K)])
            return carry

        lax.fori_loop(0, trip, step, 0)

    run(type_idx_hbm, type_tab_hbm, type_out_hbm, TYPE_SG)
    run(rel_idx_hbm, rel_tab_hbm, rel_out_hbm, REL_SG)


def kernel(x, rel_table, type_table):
    xf = x.reshape(B * L, F)
    rel_idx = xf[:, F - 1].reshape(REL_G, G)
    type_idx = xf[:, :NT].reshape(TYPE_G, G)
    rel_out, type_out = _sc_gather(rel_idx, type_idx, rel_table, type_table)
    return (rel_out.reshape(B, L, D), type_out.reshape(B, L, NT, D))


# speedup vs baseline: 5.0520x; 1.0003x over previous
import functools

import jax
import jax.numpy as jnp
from jax import lax
from jax.experimental import pallas as pl
from jax.experimental.pallas import tpu as pltpu
from jax.experimental.pallas import tpu_sc as plsc

B, L, F = 4096, 50, 10
D = 32
NT = 8
G = 128

NC, NS = 2, 16
NW = NC * NS

TYPE_G = B * L * NT // G
REL_G = B * L // G

K = 8
TYPE_SG = TYPE_G // K
REL_SG = REL_G // K
TYPE_STEPS = TYPE_SG // NW

_MESH = plsc.VectorSubcoreMesh(core_axis_name="c", subcore_axis_name="s")


@functools.partial(
    pl.kernel,
    out_type=(
        jax.ShapeDtypeStruct((REL_G, G, D), jnp.float32),
        jax.ShapeDtypeStruct((TYPE_G, G, D), jnp.float32),
    ),
    mesh=_MESH,
    compiler_params=pltpu.CompilerParams(use_tc_tiling_on_sc=False),
    scratch_types=(
        pltpu.VMEM((2, K, G), jnp.int32),
        pltpu.VMEM((2, K, G, D), jnp.float32),
        pltpu.SemaphoreType.DMA,
        pltpu.SemaphoreType.DMA,
    ),
)
def _sc_gather(rel_idx_hbm, type_idx_hbm, rel_tab_hbm, type_tab_hbm,
               rel_out_hbm, type_out_hbm, idx_v, rows_v, sem0, sem1):
    wid = lax.axis_index("s") * NC + lax.axis_index("c")
    sems = (sem0, sem1)

    def load_fire(idx_hbm, tab_hbm, slot, step):
        g = (wid + step * NW) * K
        pltpu.sync_copy(idx_hbm.at[pl.ds(g, K)], idx_v.at[slot])
        for j in range(K):
            pltpu.async_copy(
                tab_hbm.at[idx_v.at[slot].at[j]], rows_v.at[slot].at[j],
                sems[slot])

    def drain(tab_hbm, slot):
        for j in range(K):
            pltpu.make_async_copy(
                tab_hbm.at[idx_v.at[slot].at[j]], rows_v.at[slot].at[j],
                sems[slot]).wait()

    def store(out_hbm, slot, step):
        g = (wid + step * NW) * K
        pltpu.sync_copy(rows_v.at[slot], out_hbm.at[pl.ds(g, K)])

    n_pairs = TYPE_STEPS // 2
    load_fire(type_idx_hbm, type_tab_hbm, 0, 0)

    def pair(p, carry):
        s0 = 2 * p
        load_fire(type_idx_hbm, type_tab_hbm, 1, s0 + 1)
        drain(type_tab_hbm, 0)
        store(type_out_hbm, 0, s0)

        @pl.when(p < n_pairs - 1)
        def _():
            load_fire(type_idx_hbm, type_tab_hbm, 0, s0 + 2)

        drain(type_tab_hbm, 1)
        store(type_out_hbm, 1, s0 + 1)
        return carry

    lax.fori_loop(0, n_pairs, pair, 0)

    rel_trip = (REL_SG - wid + NW - 1) // NW

    def rel_step(i, carry):
        load_fire(rel_idx_hbm, rel_tab_hbm, 0, i)
        drain(rel_tab_hbm, 0)
        store(rel_out_hbm, 0, i)
        return carry

    lax.fori_loop(0, rel_trip, rel_step, 0)


def kernel(x, rel_table, type_table):
    xf = x.reshape(B * L, F)
    rel_idx = xf[:, F - 1].reshape(REL_G, G)
    type_idx = xf[:, :NT].reshape(TYPE_G, G)
    rel_out, type_out = _sc_gather(rel_idx, type_idx, rel_table, type_table)
    return (rel_out.reshape(B, L, D), type_out.reshape(B, L, NT, D))
